# Initial kernel scaffold; baseline (speedup 1.0000x reference)
import jax
import jax.numpy as jnp
from jax.experimental import pallas as pl


def kernel(x, W_e1, b_e1, W_e2, b_e2, codebook, W_d1, b_d1, W_d2, b_d2):
    raise NotImplementedError("write your pallas kernel here")


# speedup vs baseline: 1.2920x; 1.2920x over previous
import functools

import jax
import jax.numpy as jnp
from jax import lax
from jax.experimental import pallas as pl
from jax.experimental.pallas import tpu as pltpu
from jax.experimental.pallas import tpu_sc as plsc

INPUT_DIM = 768
EMBED_DIM = 256
HIDDEN = 512
NUM_EMB = 8192
BATCH = 4096
COMMIT = 0.25

BM = 512
NB = BATCH // BM

_BF = jnp.bfloat16


def _leaky(v):
    return jnp.where(v > 0, v, 0.01 * v)


def _dot(a, b):
    return jnp.dot(a.astype(_BF), b.astype(_BF),
                   preferred_element_type=jnp.float32)


def _enc_body(x_ref, we1_ref, be1_ref, we2_ref, be2_ref, ct_ref, cn_ref,
              z_ref, idx_ref):
    h = _leaky(_dot(x_ref[...], we1_ref[...]) + be1_ref[...])
    z = _dot(h, we2_ref[...]) + be2_ref[...]
    z_ref[...] = z
    znorm = jnp.sum(z * z, axis=1, keepdims=True)
    s = (znorm + cn_ref[...]) - 2.0 * _dot(z, ct_ref[...])
    idx_ref[...] = jnp.argmin(s, axis=1).astype(jnp.int32)[:, None]


def _dec_body(z_ref, zq_ref, wd1_ref, bd1_ref, wd2_ref, bd2_ref,
              xr_ref, ssq_ref):
    zq = zq_ref[...]
    h = _leaky(_dot(zq, wd1_ref[...]) + bd1_ref[...])
    xr_ref[...] = _dot(h, wd2_ref[...]) + bd2_ref[...]
    d = z_ref[...] - zq
    ssq_ref[...] = jnp.sum(d * d).reshape(1, 1, 1)


def _sc_gather(codebook, idx):
    info = plsc.get_sparse_core_info()
    nw = info.num_cores * info.num_subcores
    b_per_w = BATCH // nw
    mesh = plsc.VectorSubcoreMesh(core_axis_name="c", subcore_axis_name="s")

    @functools.partial(
        pl.kernel,
        out_type=jax.ShapeDtypeStruct((BATCH, EMBED_DIM), jnp.float32),
        mesh=mesh,
        scratch_types=[
            pltpu.VMEM((b_per_w,), jnp.int32),
            pltpu.VMEM((b_per_w, EMBED_DIM), jnp.float32),
            pltpu.SemaphoreType.DMA,
        ],
    )
    def gather_kernel(table_hbm, idx_hbm, out_hbm, idx_v, rows_v, sem):
        wid = lax.axis_index("s") * info.num_cores + lax.axis_index("c")
        base = wid * b_per_w
        pltpu.sync_copy(idx_hbm.at[pl.ds(base, b_per_w)], idx_v)
        pltpu.async_copy(table_hbm.at[idx_v], rows_v, sem).wait()
        pltpu.sync_copy(rows_v, out_hbm.at[pl.ds(base, b_per_w)])

    return gather_kernel(codebook, idx)


def kernel(x, W_e1, b_e1, W_e2, b_e2, codebook, W_d1, b_d1, W_d2, b_d2):
    ct = codebook.T
    cn = jnp.sum(codebook * codebook, axis=1)[None, :]
    full = lambda shape: pl.BlockSpec(shape, lambda i: (0,) * len(shape))

    z, idx = pl.pallas_call(
        _enc_body,
        grid=(NB,),
        in_specs=[
            pl.BlockSpec((BM, INPUT_DIM), lambda i: (i, 0)),
            full((INPUT_DIM, HIDDEN)),
            full((1, HIDDEN)),
            full((HIDDEN, EMBED_DIM)),
            full((1, EMBED_DIM)),
            full((EMBED_DIM, NUM_EMB)),
            full((1, NUM_EMB)),
        ],
        out_specs=[
            pl.BlockSpec((BM, EMBED_DIM), lambda i: (i, 0)),
            pl.BlockSpec((BM, 1), lambda i: (i, 0)),
        ],
        out_shape=[
            jax.ShapeDtypeStruct((BATCH, EMBED_DIM), jnp.float32),
            jax.ShapeDtypeStruct((BATCH, 1), jnp.int32),
        ],
        compiler_params=pltpu.CompilerParams(
            dimension_semantics=("parallel",)),
    )(x, W_e1, b_e1.reshape(1, -1), W_e2, b_e2.reshape(1, -1), ct, cn)

    zq = _sc_gather(codebook, idx.reshape(BATCH))

    xr, ssq = pl.pallas_call(
        _dec_body,
        grid=(NB,),
        in_specs=[
            pl.BlockSpec((BM, EMBED_DIM), lambda i: (i, 0)),
            pl.BlockSpec((BM, EMBED_DIM), lambda i: (i, 0)),
            full((EMBED_DIM, HIDDEN)),
            full((1, HIDDEN)),
            full((HIDDEN, INPUT_DIM)),
            full((1, INPUT_DIM)),
        ],
        out_specs=[
            pl.BlockSpec((BM, INPUT_DIM), lambda i: (i, 0)),
            pl.BlockSpec((1, 1, 1), lambda i: (i, 0, 0)),
        ],
        out_shape=[
            jax.ShapeDtypeStruct((BATCH, INPUT_DIM), jnp.float32),
            jax.ShapeDtypeStruct((NB, 1, 1), jnp.float32),
        ],
        compiler_params=pltpu.CompilerParams(
            dimension_semantics=("parallel",)),
    )(z, zq, W_d1, b_d1.reshape(1, -1), W_d2, b_d2.reshape(1, -1))

    vq_loss = (1.0 + COMMIT) * jnp.sum(ssq) / (BATCH * EMBED_DIM)
    return xr, vq_loss
